# Initial kernel scaffold; baseline (speedup 1.0000x reference)
import jax
import jax.numpy as jnp
from jax.experimental import pallas as pl


def kernel(x, edge_index, batch, W1, att_src, att_dst, b1, W2, b2, Wfc1, bfc1, Wfc2, bfc2):
    raise NotImplementedError("write your pallas kernel here")


# speedup vs baseline: 1.0344x; 1.0344x over previous
import functools
import jax
import jax.numpy as jnp
from jax.experimental import pallas as pl
from jax.experimental.pallas import tpu as pltpu

N = 10000
E = 160000
F = 128
H = 10
HID = 1280
G = 256
FC1 = 1500
OUT = 128
BLK = 512
HPAD = 16


def _k1_body(x_ref, w1_ref, asrc_ref, adst_ref, *out_refs):
    hb = jnp.dot(x_ref[...], w1_ref[...], preferred_element_type=jnp.float32)
    for p in range(H):
        out_refs[p][...] = hb[:, p * F:(p + 1) * F]
    out_refs[H][...] = jnp.dot(hb, asrc_ref[...], preferred_element_type=jnp.float32)
    out_refs[H + 1][...] = jnp.dot(hb, adst_ref[...], preferred_element_type=jnp.float32)


def _k1(x, W1, A_src, A_dst):
    grid = (pl.cdiv(N, BLK),)
    outs = [jax.ShapeDtypeStruct((N, F), jnp.float32) for _ in range(H)]
    outs += [jax.ShapeDtypeStruct((N, HPAD), jnp.float32)] * 2
    in_specs = [
        pl.BlockSpec((BLK, F), lambda i: (i, 0)),
        pl.BlockSpec((F, HID), lambda i: (0, 0)),
        pl.BlockSpec((HID, HPAD), lambda i: (0, 0)),
        pl.BlockSpec((HID, HPAD), lambda i: (0, 0)),
    ]
    out_specs = [pl.BlockSpec((BLK, F), lambda i: (i, 0)) for _ in range(H)]
    out_specs += [pl.BlockSpec((BLK, HPAD), lambda i: (i, 0))] * 2
    return pl.pallas_call(
        _k1_body, grid=grid, in_specs=in_specs, out_specs=out_specs,
        out_shape=outs)(x, W1, A_src, A_dst)


def _k3_body(x1_ref, w2_ref, *out_refs):
    hb = jnp.dot(x1_ref[...], w2_ref[...], preferred_element_type=jnp.float32)
    for p in range(H):
        out_refs[p][...] = hb[:, p * F:(p + 1) * F]


def _k3(x1, W2):
    grid = (pl.cdiv(N, BLK),)
    outs = [jax.ShapeDtypeStruct((N, F), jnp.float32) for _ in range(H)]
    in_specs = [
        pl.BlockSpec((BLK, HID), lambda i: (i, 0)),
        pl.BlockSpec((HID, HID), lambda i: (0, 0)),
    ]
    out_specs = [pl.BlockSpec((BLK, F), lambda i: (i, 0)) for _ in range(H)]
    return pl.pallas_call(
        _k3_body, grid=grid, in_specs=in_specs, out_specs=out_specs,
        out_shape=outs)(x1, W2)


def _k7_body(gmax_ref, gmean_ref, wfc1_ref, bfc1_ref, wfc2_ref, bfc2_ref, y_ref):
    g = jnp.concatenate([gmax_ref[...], gmean_ref[...]], axis=1)
    t = jnp.dot(g, wfc1_ref[...], preferred_element_type=jnp.float32) + bfc1_ref[...]
    t = jnp.maximum(t, 0.0)
    y_ref[...] = jnp.dot(t, wfc2_ref[...], preferred_element_type=jnp.float32) + bfc2_ref[...]


def _k7(gmax, gmean, Wfc1, bfc1, Wfc2, bfc2):
    return pl.pallas_call(
        _k7_body,
        out_shape=jax.ShapeDtypeStruct((G, OUT), jnp.float32),
    )(gmax, gmean, Wfc1, bfc1.reshape(1, FC1), Wfc2, bfc2.reshape(1, OUT))


def kernel(x, edge_index, batch, W1, att_src, att_dst, b1, W2, b2, Wfc1, bfc1, Wfc2, bfc2):
    eye = jnp.eye(HPAD, dtype=jnp.float32)[:H]
    A_src = (att_src[0][:, :, None] * eye[:, None, :]).reshape(HID, HPAD)
    A_dst = (att_dst[0][:, :, None] * eye[:, None, :]).reshape(HID, HPAD)

    outs = _k1(x, W1, A_src, A_dst)
    hs, a_s, a_d = outs[:H], outs[H][:, :H], outs[H + 1][:, :H]
    h = jnp.stack(hs, axis=1)

    n = x.shape[0]
    loop = jnp.arange(n, dtype=edge_index.dtype)
    src = jnp.concatenate([edge_index[0], loop])
    dst = jnp.concatenate([edge_index[1], loop])

    e = jax.nn.leaky_relu(a_s[src] + a_d[dst], negative_slope=0.2)
    ex = jnp.exp(e)
    den = jax.ops.segment_sum(ex, dst, num_segments=n)
    msg = h[src] * (ex / (den[dst] + 1e-16))[:, :, None]
    x1 = jax.ops.segment_sum(msg, dst, num_segments=n).reshape(n, HID) + b1
    x1 = jax.nn.relu(x1)

    deg = jax.ops.segment_sum(jnp.ones(src.shape[0], dtype=x.dtype), dst, num_segments=n)
    dinv = jnp.where(deg > 0, jax.lax.rsqrt(jnp.maximum(deg, 1e-12)), 0.0)
    norm = dinv[src] * dinv[dst]
    h2 = jnp.concatenate(_k3(x1, W2), axis=1)
    x2 = jax.ops.segment_sum(h2[src] * norm[:, None], dst, num_segments=n) + b2
    x2 = jax.nn.relu(x2)

    gmax = jax.ops.segment_max(x2, batch, num_segments=G)
    gmax = jnp.where(jnp.isfinite(gmax), gmax, 0.0)
    cnt = jax.ops.segment_sum(jnp.ones((n,), dtype=x.dtype), batch, num_segments=G)
    gsum = jax.ops.segment_sum(x2, batch, num_segments=G)
    gmean = gsum / jnp.maximum(cnt, 1.0)[:, None]

    return _k7(gmax, gmean, Wfc1, bfc1, Wfc2, bfc2)


# speedup vs baseline: 1.4570x; 1.4086x over previous
import jax
import jax.numpy as jnp
from jax import lax
from jax.experimental import pallas as pl
from jax.experimental.pallas import tpu as pltpu
from jax.experimental.pallas import tpu_sc as plsc

N = 10000
E = 160000
F = 128
H = 10
HID = 1280
G = 256
FC1 = 1500
OUT = 128
BLK = 512

EPT = 10016
NG = EPT // 16
EPAD = 16 * EPT
NLOC = 5120
NACC = NLOC + 16
RPT = NLOC // 16
NG2 = 10240


def _k1_body(x_ref, w1_ref, asrc_ref, adst_ref, *out_refs):
    hb = jnp.dot(x_ref[...], w1_ref[...], preferred_element_type=jnp.float32)
    for p in range(H):
        out_refs[p][...] = hb[:, p * F:(p + 1) * F]
    out_refs[H][...] = jnp.exp(
        jnp.dot(hb, asrc_ref[...], preferred_element_type=jnp.float32))
    out_refs[H + 1][...] = jnp.exp(
        jnp.dot(hb, adst_ref[...], preferred_element_type=jnp.float32))


def _k1(x, W1, A_src, A_dst):
    grid = (pl.cdiv(N, BLK),)
    outs = [jax.ShapeDtypeStruct((N, F), jnp.float32) for _ in range(H + 2)]
    in_specs = [
        pl.BlockSpec((BLK, F), lambda i: (i, 0)),
        pl.BlockSpec((F, HID), lambda i: (0, 0)),
        pl.BlockSpec((HID, F), lambda i: (0, 0)),
        pl.BlockSpec((HID, F), lambda i: (0, 0)),
    ]
    out_specs = [pl.BlockSpec((BLK, F), lambda i: (i, 0)) for _ in range(H + 2)]
    return pl.pallas_call(
        _k1_body, grid=grid, in_specs=in_specs, out_specs=out_specs,
        out_shape=outs)(x, W1, A_src, A_dst)


_MESH = plsc.VectorSubcoreMesh(core_axis_name="c", subcore_axis_name="s",
                               num_cores=2)

_GDN = lax.GatherDimensionNumbers(
    offset_dims=(), collapsed_slice_dims=(0,), start_index_map=(0,))


def _splat(v, k):
    idx = jnp.full((16, 1), k, jnp.int32)
    return lax.gather(v, idx, _GDN, (1,),
                      mode=lax.GatherScatterMode.PROMISE_IN_BOUNDS)


def _init_zero_buf(zeros):
    def zb(i, carry):
        for cc in range(8):
            zeros[i, pl.ds(cc * 16, 16)] = jnp.zeros((16,), jnp.float32)
        return carry
    lax.fori_loop(0, 64, zb, 0)


def _zero_rows(zeros, accum, s):
    for b in range(5):
        pltpu.sync_copy(zeros.at[pl.ds(0, 64)],
                        accum.at[pl.ds(s * RPT + b * 64, 64)])

    @pl.when(s == 0)
    def _():
        pltpu.sync_copy(zeros.at[pl.ds(0, 16)], accum.at[pl.ds(NLOC, 16)])


def _writeback(accum, out, c, s):
    src = s * RPT
    dst = pl.multiple_of(c * NLOC + s * RPT, 8)
    pltpu.sync_copy(accum.at[pl.ds(src, RPT)], out.at[pl.ds(dst, RPT)])


def _lidx(dstv, c):
    l = dstv - c * NLOC
    inb = (l >= 0) & (l < NLOC)
    return jnp.where(inb, l, NLOC)


def _k2_body(srcf, dstf, au_t, av_t, *rest):
    hs = rest[:H]
    un_out, den_out = rest[H], rest[H + 1]
    (src_v, dst_v, extile, rows, arow, brow, zeros, accum) = rest[H + 2:]
    c = lax.axis_index("c")
    s = lax.axis_index("s")

    _init_zero_buf(zeros)
    def zext(i, carry):
        for cc in range(8):
            extile[i, pl.ds(cc * 16, 16)] = jnp.zeros((16,), jnp.float32)
        return carry
    lax.fori_loop(0, 16, zext, 0)
    pltpu.sync_copy(srcf.at[pl.ds(s * EPT, EPT)], src_v)
    pltpu.sync_copy(dstf.at[pl.ds(s * EPT, EPT)], dst_v)
    _zero_rows(zeros, accum, s)
    plsc.subcore_barrier()

    def exbody(j, carry):
        sidx = src_v[pl.ds(j * 16, 16)]
        didx = dst_v[pl.ds(j * 16, 16)]
        pltpu.sync_copy(au_t.at[sidx], arow)
        pltpu.sync_copy(av_t.at[didx], brow)
        for k in range(16):
            e1 = arow[k, pl.ds(0, 16)] * brow[k, pl.ds(0, 16)]
            e2 = arow[k, pl.ds(16, 16)] * brow[k, pl.ds(16, 16)]
            extile[k, pl.ds(0, 16)] = jnp.maximum(e1, e2)
        pltpu.sync_copy(extile, accum.at[_lidx(didx, c)], add=True)
        return carry
    lax.fori_loop(0, NG, exbody, 0)
    plsc.subcore_barrier()
    _writeback(accum, den_out, c, s)

    for p in range(H):
        _zero_rows(zeros, accum, s)
        plsc.subcore_barrier()

        def gbody(j, carry, p=p):
            sidx = src_v[pl.ds(j * 16, 16)]
            didx = dst_v[pl.ds(j * 16, 16)]
            pltpu.sync_copy(hs[p].at[sidx], rows)
            pltpu.sync_copy(au_t.at[sidx], arow)
            pltpu.sync_copy(av_t.at[didx], brow)
            for k in range(16):
                e1 = arow[k, pl.ds(0, 16)] * brow[k, pl.ds(0, 16)]
                e2 = arow[k, pl.ds(16, 16)] * brow[k, pl.ds(16, 16)]
                spl = _splat(jnp.maximum(e1, e2), p)
                for cc in range(8):
                    rows[k, pl.ds(cc * 16, 16)] = rows[k, pl.ds(cc * 16, 16)] * spl
            pltpu.sync_copy(rows, accum.at[_lidx(didx, c)], add=True)
            return carry
        lax.fori_loop(0, NG, gbody, 0)
        plsc.subcore_barrier()
        _writeback(accum, un_out.at[p], c, s)


_k2 = pl.kernel(
    _k2_body, mesh=_MESH,
    out_type=[jax.ShapeDtypeStruct((H, NG2, F), jnp.float32),
              jax.ShapeDtypeStruct((NG2, F), jnp.float32)],
    scratch_types=[
        pltpu.VMEM((EPT,), jnp.int32),
        pltpu.VMEM((EPT,), jnp.int32),
        pltpu.VMEM((16, F), jnp.float32),
        pltpu.VMEM((16, F), jnp.float32),
        pltpu.VMEM((16, F), jnp.float32),
        pltpu.VMEM((16, F), jnp.float32),
        pltpu.VMEM((64, F), jnp.float32),
        pltpu.VMEM_SHARED((NACC, F), jnp.float32),
    ])


def _k4_body(srcf, dstf, *rest):
    hs = rest[:H]
    un_out = rest[H]
    (src_v, dst_v, rows, zeros, accum) = rest[H + 1:]
    c = lax.axis_index("c")
    s = lax.axis_index("s")

    _init_zero_buf(zeros)
    pltpu.sync_copy(srcf.at[pl.ds(s * EPT, EPT)], src_v)
    pltpu.sync_copy(dstf.at[pl.ds(s * EPT, EPT)], dst_v)

    for p in range(H):
        _zero_rows(zeros, accum, s)
        plsc.subcore_barrier()

        def gbody(j, carry, p=p):
            sidx = src_v[pl.ds(j * 16, 16)]
            didx = dst_v[pl.ds(j * 16, 16)]
            pltpu.sync_copy(hs[p].at[sidx], rows)
            pltpu.sync_copy(rows, accum.at[_lidx(didx, c)], add=True)
            return carry
        lax.fori_loop(0, NG, gbody, 0)
        plsc.subcore_barrier()
        _writeback(accum, un_out.at[p], c, s)


_k4 = pl.kernel(
    _k4_body, mesh=_MESH,
    out_type=[jax.ShapeDtypeStruct((H, NG2, F), jnp.float32)],
    scratch_types=[
        pltpu.VMEM((EPT,), jnp.int32),
        pltpu.VMEM((EPT,), jnp.int32),
        pltpu.VMEM((16, F), jnp.float32),
        pltpu.VMEM((64, F), jnp.float32),
        pltpu.VMEM_SHARED((NACC, F), jnp.float32),
    ])


def _k3_body(un_ref, den_ref, au_ref, av_ref, *rest):
    h_refs = rest[:H]
    b1_ref, w2_ref = rest[H], rest[H + 1]
    out_refs = rest[H + 2:]
    au, av = au_ref[...], av_ref[...]
    ex_self = jnp.maximum(au[:, 0:16] * av[:, 0:16],
                          au[:, 16:32] * av[:, 16:32])
    den_tot = den_ref[:, 0:16] + ex_self
    rden = 1.0 / den_tot
    cols = []
    for p in range(H):
        u = un_ref[p] + ex_self[:, p:p + 1] * h_refs[p][...]
        cols.append(jnp.maximum(u * rden[:, p:p + 1] + b1_ref[p:p + 1, :], 0.0))
    x1 = jnp.concatenate(cols, axis=1)
    hb = jnp.dot(x1, w2_ref[...], preferred_element_type=jnp.float32)
    dinv = lax.rsqrt(den_tot[:, 15:16])
    for p in range(H):
        out_refs[p][...] = dinv * hb[:, p * F:(p + 1) * F]
    out_refs[H][...] = jnp.broadcast_to(dinv, (BLK, F))


def _k3(un, den, au, av, hs, b1, W2):
    grid = (pl.cdiv(N, BLK),)
    outs = [jax.ShapeDtypeStruct((N, F), jnp.float32) for _ in range(H + 1)]
    in_specs = [
        pl.BlockSpec((H, BLK, F), lambda i: (0, i, 0)),
        pl.BlockSpec((BLK, F), lambda i: (i, 0)),
        pl.BlockSpec((BLK, F), lambda i: (i, 0)),
        pl.BlockSpec((BLK, F), lambda i: (i, 0)),
    ]
    in_specs += [pl.BlockSpec((BLK, F), lambda i: (i, 0)) for _ in range(H)]
    in_specs += [
        pl.BlockSpec((H, F), lambda i: (0, 0)),
        pl.BlockSpec((HID, HID), lambda i: (0, 0)),
    ]
    out_specs = [pl.BlockSpec((BLK, F), lambda i: (i, 0)) for _ in range(H + 1)]
    return pl.pallas_call(
        _k3_body, grid=grid, in_specs=in_specs, out_specs=out_specs,
        out_shape=outs)(un, den, au, av, *hs, b1, W2)


def _k5_body(un_ref, dinv_ref, *rest):
    h_refs = rest[:H]
    b2_ref = rest[H]
    out_refs = rest[H + 1:]
    dv = dinv_ref[:, 0:1]
    for p in range(H):
        t = (un_ref[p] + h_refs[p][...]) * dv
        out_refs[p][...] = jnp.maximum(t + b2_ref[p:p + 1, :], 0.0)


def _k5(un2, dinv, h2s, b2):
    grid = (pl.cdiv(N, BLK),)
    outs = [jax.ShapeDtypeStruct((N, F), jnp.float32) for _ in range(H)]
    in_specs = [
        pl.BlockSpec((H, BLK, F), lambda i: (0, i, 0)),
        pl.BlockSpec((BLK, F), lambda i: (i, 0)),
    ]
    in_specs += [pl.BlockSpec((BLK, F), lambda i: (i, 0)) for _ in range(H)]
    in_specs += [pl.BlockSpec((H, F), lambda i: (0, 0))]
    out_specs = [pl.BlockSpec((BLK, F), lambda i: (i, 0)) for _ in range(H)]
    return pl.pallas_call(
        _k5_body, grid=grid, in_specs=in_specs, out_specs=out_specs,
        out_shape=outs)(un2, dinv, *h2s, b2)


def _k7_body(gmax_ref, gmean_ref, wfc1_ref, bfc1_ref, wfc2_ref, bfc2_ref, y_ref):
    g = jnp.concatenate([gmax_ref[...], gmean_ref[...]], axis=1)
    t = jnp.dot(g, wfc1_ref[...], preferred_element_type=jnp.float32) + bfc1_ref[...]
    t = jnp.maximum(t, 0.0)
    y_ref[...] = jnp.dot(t, wfc2_ref[...], preferred_element_type=jnp.float32) + bfc2_ref[...]


def _k7(gmax, gmean, Wfc1, bfc1, Wfc2, bfc2):
    return pl.pallas_call(
        _k7_body,
        out_shape=jax.ShapeDtypeStruct((G, OUT), jnp.float32),
    )(gmax, gmean, Wfc1, bfc1.reshape(1, FC1), Wfc2, bfc2.reshape(1, OUT))


def kernel(x, edge_index, batch, W1, att_src, att_dst, b1, W2, b2, Wfc1, bfc1, Wfc2, bfc2):
    sel = (jnp.eye(F, dtype=jnp.float32)[:H]
           + 0.2 * jnp.eye(F, dtype=jnp.float32)[16:16 + H])
    A_src = (att_src[0][:, :, None] * sel[:, None, :]).reshape(HID, F)
    A_dst = (att_dst[0][:, :, None] * sel[:, None, :]).reshape(HID, F)

    outs = _k1(x, W1, A_src, A_dst)
    hs, au, av = list(outs[:H]), outs[H], outs[H + 1]

    pad = EPAD - E
    srcf = jnp.concatenate([edge_index[0], jnp.zeros((pad,), jnp.int32)])
    dstf = jnp.concatenate([edge_index[1], jnp.full((pad,), N, jnp.int32)])

    un, den = _k2(srcf, dstf, au, av, *hs)
    k3_outs = _k3(un, den, au, av, hs, b1.reshape(H, F), W2)
    h2s, dinv = list(k3_outs[:H]), k3_outs[H]

    (un2,) = _k4(srcf, dstf, *h2s)
    x2s = _k5(un2, dinv, h2s, b2.reshape(H, F))
    x2 = jnp.concatenate(x2s, axis=1)

    n = x.shape[0]
    gmax = jax.ops.segment_max(x2, batch, num_segments=G)
    gmax = jnp.where(jnp.isfinite(gmax), gmax, 0.0)
    cnt = jax.ops.segment_sum(jnp.ones((n,), dtype=x.dtype), batch, num_segments=G)
    gsum = jax.ops.segment_sum(x2, batch, num_segments=G)
    gmean = gsum / jnp.maximum(cnt, 1.0)[:, None]

    return _k7(gmax, gmean, Wfc1, bfc1, Wfc2, bfc2)
